# baseline (speedup 1.0000x reference)
import functools

import jax
import jax.numpy as jnp
from jax import lax
from jax.experimental import pallas as pl
from jax.experimental.pallas import tpu as pltpu
from jax.experimental.pallas import tpu_sc as plsc

N_GRAPHS = 500
N_PER = 20
TOTAL = N_GRAPHS * N_PER
E = 320000
D = 128
HD = 128
L = 4
NUM_HEADS = 4
N_ACT = 56
EPS = 1e-5

GP = 512
NP = 10112
NPAD_ROWS = NP - TOTAL
NC = 2
NS = 16
NW = NC * NS
CHUNK = 128
CH = 81
E_PAD = NW * CH * CHUNK
ROWS_PER_TEC = NP // NS

_HIGH = jax.lax.Precision.HIGHEST


def _relu(v):
    return jnp.maximum(v, 0.0)


def _dot(a, b):
    return jax.lax.dot_general(a, b, (((1,), (0,)), ((), ())),
                               preferred_element_type=jnp.float32,
                               precision=_HIGH)


def _zero_block(ref, rows, width):
    z = jnp.zeros((16,), jnp.float32)

    def body(r, _):
        for q in range(width // 16):
            ref[r, pl.ds(q * 16, 16)] = z
        return 0

    lax.fori_loop(0, rows, body, 0)


def _zero_rows(zsrc, dst_sh, my_rows):
    nfull = ROWS_PER_TEC // CHUNK
    rem = ROWS_PER_TEC - nfull * CHUNK
    for k in range(nfull):
        pltpu.sync_copy(zsrc, dst_sh.at[pl.ds(my_rows + k * CHUNK, CHUNK)])
    if rem:
        pltpu.sync_copy(
            zsrc.at[pl.ds(0, rem)],
            dst_sh.at[pl.ds(my_rows + nfull * CHUNK, rem)])


def _make_sc_pass():
    outs = [jax.ShapeDtypeStruct((NC * NP, D), jnp.float32)]
    scratch = [
        [pltpu.VMEM((CHUNK,), jnp.int32) for _ in range(3)],
        [pltpu.VMEM((CHUNK,), jnp.int32) for _ in range(3)],
        [pltpu.VMEM((CHUNK, D), jnp.float32) for _ in range(3)],
        pltpu.VMEM_SHARED((NP, D), jnp.float32),
        [pltpu.SemaphoreType.DMA for _ in range(3)],
        [pltpu.SemaphoreType.DMA for _ in range(3)],
    ]

    def body(h_hbm, src_hbm, dst_hbm, agg_out, src_v, dst_v, rows, agg_sh,
             sem_g, sem_s):
        c = lax.axis_index("c")
        s = lax.axis_index("s")
        wid = c * NS + s
        my_rows = s * ROWS_PER_TEC
        ebase = wid * CH * CHUNK

        _zero_block(rows[0], CHUNK, D)
        _zero_rows(rows[0], agg_sh, my_rows)
        plsc.subcore_barrier()

        def load_idx(i, k):
            pltpu.sync_copy(src_hbm.at[pl.ds(ebase + i * CHUNK, CHUNK)],
                            src_v[k])
            pltpu.sync_copy(dst_hbm.at[pl.ds(ebase + i * CHUNK, CHUNK)],
                            dst_v[k])

        def fire_g(k):
            pltpu.async_copy(h_hbm.at[src_v[k]], rows[k], sem_g[k])

        def wait_g(k):
            pltpu.make_async_copy(h_hbm.at[src_v[k]], rows[k],
                                  sem_g[k]).wait()

        def fire_s(k):
            pltpu.async_copy(rows[k], agg_sh.at[dst_v[k]], sem_s[k],
                             add=True)

        def wait_s(k):
            pltpu.make_async_copy(rows[k], agg_sh.at[dst_v[k]],
                                  sem_s[k]).wait()

        load_idx(0, 0)
        fire_g(0)
        load_idx(1, 1)
        fire_g(1)
        wait_g(0)
        fire_s(0)
        load_idx(2, 2)
        fire_g(2)
        wait_g(1)
        fire_s(1)

        def step(t, _):
            for k in range(3):
                i = 3 * t + k
                wait_s(k)
                load_idx(i, k)
                fire_g(k)
                kp = (k + 2) % 3
                wait_g(kp)
                fire_s(kp)
            return 0

        lax.fori_loop(1, CH // 3, step, 0)
        wait_g(2)
        fire_s(2)
        wait_s(0)
        wait_s(1)
        wait_s(2)
        plsc.subcore_barrier()

        base = c * NP + my_rows
        pltpu.sync_copy(agg_sh.at[pl.ds(my_rows, ROWS_PER_TEC)],
                        agg_out.at[pl.ds(base, ROWS_PER_TEC)])

    mesh = plsc.VectorSubcoreMesh(core_axis_name="c", subcore_axis_name="s",
                                  num_cores=NC, num_subcores=NS)
    return pl.kernel(body, out_type=outs, mesh=mesh, scratch_types=scratch)


DW = 128


def _make_sc_deg():
    outs = [jax.ShapeDtypeStruct((NC * NP, DW), jnp.float32)]
    scratch = [
        pltpu.VMEM((CHUNK,), jnp.int32),
        pltpu.VMEM((CHUNK,), jnp.int32),
        pltpu.VMEM((CHUNK, DW), jnp.float32),
        pltpu.VMEM_SHARED((NP, DW), jnp.float32),
        pltpu.SemaphoreType.DMA,
        pltpu.SemaphoreType.DMA,
    ]

    def body(dst_hbm, degb_out, dst0, dst1, ones_v, deg_sh, sem_a, sem_b):
        c = lax.axis_index("c")
        s = lax.axis_index("s")
        wid = c * NS + s
        my_rows = s * ROWS_PER_TEC
        ebase = wid * CH * CHUNK

        _zero_block(ones_v, CHUNK, DW)
        _zero_rows(ones_v, deg_sh, my_rows)
        one = jnp.ones((16,), jnp.float32)

        def fill_ones(r, _):
            for q in range(DW // 16):
                ones_v[r, pl.ds(q * 16, 16)] = one
            return 0

        lax.fori_loop(0, CHUNK, fill_ones, 0)
        plsc.subcore_barrier()

        pltpu.sync_copy(dst_hbm.at[pl.ds(ebase, CHUNK)], dst0)
        pltpu.async_copy(ones_v, deg_sh.at[dst0], sem_a, add=True)

        def step(t, _):
            i = 2 * t
            pltpu.sync_copy(dst_hbm.at[pl.ds(ebase + (i + 1) * CHUNK, CHUNK)],
                            dst1)
            pltpu.async_copy(ones_v, deg_sh.at[dst1], sem_b, add=True)
            pltpu.make_async_copy(ones_v, deg_sh.at[dst0], sem_a).wait()
            pltpu.sync_copy(dst_hbm.at[pl.ds(ebase + (i + 2) * CHUNK, CHUNK)],
                            dst0)
            pltpu.async_copy(ones_v, deg_sh.at[dst0], sem_a, add=True)
            pltpu.make_async_copy(ones_v, deg_sh.at[dst1], sem_b).wait()
            return 0

        lax.fori_loop(0, CH // 2, step, 0)
        pltpu.make_async_copy(ones_v, deg_sh.at[dst0], sem_a).wait()
        plsc.subcore_barrier()

        base = c * NP + my_rows
        pltpu.sync_copy(deg_sh.at[pl.ds(my_rows, ROWS_PER_TEC)],
                        degb_out.at[pl.ds(base, ROWS_PER_TEC)])

    mesh = plsc.VectorSubcoreMesh(core_axis_name="c", subcore_axis_name="s",
                                  num_cores=NC, num_subcores=NS)
    return pl.kernel(body, out_type=outs, mesh=mesh, scratch_types=scratch)


@functools.cache
def _get_sc_pass():
    return _make_sc_pass()


@functools.cache
def _get_sc_deg():
    return _make_sc_deg()


def _sc_pass(h, src_slab, dst_slab):
    out = _get_sc_pass()(h, src_slab, dst_slab)
    return out[0] if isinstance(out, (list, tuple)) else out


def _sc_deg(dst_slab):
    out = _get_sc_deg()(dst_slab)
    return out[0] if isinstance(out, (list, tuple)) else out


_BLK = 1264
_GRID = NP // _BLK

_row_spec = pl.BlockSpec((_BLK, D), lambda i: (i, 0))
_half0_spec = pl.BlockSpec((_BLK, D), lambda i: (i, 0))
_half1_spec = pl.BlockSpec((_BLK, D), lambda i: (i + _GRID, 0))
_w_spec = pl.BlockSpec((D, D), lambda i: (0, 0))
_b_spec = pl.BlockSpec((1, D), lambda i: (0, 0))


def _proj_body(x_ref, w_ref, b_ref, o_ref):
    o_ref[...] = _relu(_dot(x_ref[...], w_ref[...]) + b_ref[...])


_proj = pl.pallas_call(
    _proj_body,
    grid=(_GRID,),
    in_specs=[_row_spec, _w_spec, _b_spec],
    out_specs=_row_spec,
    out_shape=jax.ShapeDtypeStruct((NP, D), jnp.float32),
)


def _layer1_body(a0_ref, a1_ref, d0_ref, d1_ref, h_ref, w_ref, b_ref,
                 o_ref, rec_ref):
    rec64 = 1.0 / jnp.maximum(d0_ref[...] + d1_ref[...], 1.0)
    rec = jnp.broadcast_to(rec64[:, :1], (_BLK, D))
    rec_ref[...] = rec
    a = (a0_ref[...] + a1_ref[...]) * rec
    o_ref[...] = _relu(_dot(a, w_ref[...]) + b_ref[...]) + h_ref[...]


_deg0_spec = pl.BlockSpec((_BLK, DW), lambda i: (i, 0))
_deg1_spec = pl.BlockSpec((_BLK, DW), lambda i: (i + _GRID, 0))

_layer1 = pl.pallas_call(
    _layer1_body,
    grid=(_GRID,),
    in_specs=[_half0_spec, _half1_spec, _deg0_spec, _deg1_spec, _row_spec,
              _w_spec, _b_spec],
    out_specs=[_row_spec, _row_spec],
    out_shape=[jax.ShapeDtypeStruct((NP, D), jnp.float32),
               jax.ShapeDtypeStruct((NP, D), jnp.float32)],
)


def _layer_body(a0_ref, a1_ref, rec_ref, h_ref, w_ref, b_ref, o_ref):
    a = (a0_ref[...] + a1_ref[...]) * rec_ref[...]
    o_ref[...] = _relu(_dot(a, w_ref[...]) + b_ref[...]) + h_ref[...]


_layer = pl.pallas_call(
    _layer_body,
    grid=(_GRID,),
    in_specs=[_half0_spec, _half1_spec, _row_spec, _row_spec, _w_spec,
              _b_spec],
    out_specs=_row_spec,
    out_shape=jax.ShapeDtypeStruct((NP, D), jnp.float32),
)


def _heads_body(h3_ref, we_ref, be_ref, lng_ref, lnb_ref, wh_ref, bh_ref,
                wp1_ref, bp1_ref, wp2_ref, bp2_ref, wv1_ref, bv1_ref,
                wv2_ref, bv2_ref, l0_ref, l1_ref, l2_ref, l3_ref, v_ref):
    hb = h3_ref[...]
    be = be_ref[...]
    lng = lng_ref[...]
    lnb = lnb_ref[...]
    bp1 = bp1_ref[...]
    bp2 = bp2_ref[...]

    embs = []
    for i in range(NUM_HEADS):
        base = 4 * i
        acc = hb[:, base, :]
        mx = hb[:, base, :]
        for j in range(1, 8):
            t = hb[:, base + j, :]
            acc = acc + t
            mx = jnp.maximum(mx, t)
        m = acc * (1.0 / 8.0)
        y = (_dot(m, we_ref[i, 0:D, :]) + _dot(mx, we_ref[i, D:2 * D, :])
             + be[i:i + 1, :])
        mu = jnp.mean(y, axis=-1, keepdims=True)
        var = jnp.mean((y - mu) ** 2, axis=-1, keepdims=True)
        y = (y - mu) / jnp.sqrt(var + EPS) * lng[i:i + 1, :] + lnb[i:i + 1, :]
        embs.append(_relu(y))

    glob = (embs[0] + embs[1] + embs[2] + embs[3]) * 0.25
    ctx = _relu(_dot(glob, wh_ref[...]) + bh_ref[...])

    outs = [l0_ref, l1_ref, l2_ref, l3_ref]
    for i in range(NUM_HEADS):
        z = _relu(_dot(embs[i], wp1_ref[i, 0:HD, :])
                  + _dot(ctx, wp1_ref[i, HD:2 * HD, :]) + bp1[i:i + 1, :])
        outs[i][...] = _dot(z, wp2_ref[i]) + bp2[i:i + 1, :]

    pv = hb[:, 0, :]
    for j in range(1, N_PER):
        pv = pv + hb[:, j, :]
    pv = pv * (1.0 / N_PER)
    v = _relu(_dot(pv, wv1_ref[...]) + bv1_ref[...])
    v_ref[...] = jnp.tanh(_dot(v, wv2_ref[...]) + bv2_ref[...])


_GB = 128
_HGRID = GP // _GB


def _full(shape):
    return pl.BlockSpec(shape, lambda i: tuple(0 for _ in shape))


_heads = pl.pallas_call(
    _heads_body,
    grid=(_HGRID,),
    in_specs=[pl.BlockSpec((_GB, N_PER, D), lambda i: (i, 0, 0)),
              _full((NUM_HEADS, 2 * D, HD)),
              _full((NUM_HEADS, HD)), _full((NUM_HEADS, HD)),
              _full((NUM_HEADS, HD)), _full((HD, HD)), _full((1, HD)),
              _full((NUM_HEADS, 2 * HD, HD)), _full((NUM_HEADS, HD)),
              _full((NUM_HEADS, HD, N_ACT)), _full((NUM_HEADS, N_ACT)),
              _full((D, D)), _full((1, D)), _full((D, 1)), _full((1, 1))],
    out_specs=[pl.BlockSpec((_GB, N_ACT), lambda i: (i, 0))] * NUM_HEADS
    + [pl.BlockSpec((_GB, 1), lambda i: (i, 0))],
    out_shape=[jax.ShapeDtypeStruct((GP, N_ACT), jnp.float32)] * NUM_HEADS
    + [jax.ShapeDtypeStruct((GP, 1), jnp.float32)],
)


def kernel(x, edge_index, batch, W_in, b_in, W_layers, b_layers, W_ext, b_ext,
           ln_g, ln_b, W_hub, b_hub, W_p1, b_p1, W_p2, b_p2, W_v1, b_v1,
           W_v2, b_v2):
    del batch

    src = edge_index[0]
    dst = edge_index[1]
    pad = TOTAL + (jnp.arange(E_PAD - E, dtype=jnp.int32) % NPAD_ROWS)
    src_slab = jnp.concatenate([src, pad])
    dst_slab = jnp.concatenate([dst, pad])

    x_pad = jnp.pad(x, ((0, NP - TOTAL), (0, 0)))
    b_in2 = b_in.reshape(1, D)

    h = _proj(x_pad, W_in, b_in2)

    degb = _sc_deg(dst_slab)
    aggs = _sc_pass(h, src_slab, dst_slab)
    h, rec = _layer1(aggs, aggs, degb, degb, h,
                     W_layers[0], b_layers[0].reshape(1, D))
    for l in range(1, L):
        aggs = _sc_pass(h, src_slab, dst_slab)
        h = _layer(aggs, aggs, rec, h, W_layers[l],
                   b_layers[l].reshape(1, D))

    h3 = jnp.pad(h, ((0, GP * N_PER - NP), (0, 0))).reshape(GP, N_PER, D)
    l0, l1, l2, l3, v = _heads(
        h3, W_ext, b_ext, ln_g, ln_b, W_hub, b_hub.reshape(1, HD),
        W_p1, b_p1, W_p2, b_p2, W_v1, b_v1.reshape(1, D), W_v2,
        b_v2.reshape(1, 1))

    return jnp.concatenate(
        [l0[:N_GRAPHS], l1[:N_GRAPHS], l2[:N_GRAPHS], l3[:N_GRAPHS],
         v[:N_GRAPHS]], axis=-1)


# speedup vs baseline: 1.0056x; 1.0056x over previous
import functools

import jax
import jax.numpy as jnp
from jax import lax
from jax.experimental import pallas as pl
from jax.experimental.pallas import tpu as pltpu
from jax.experimental.pallas import tpu_sc as plsc

N_GRAPHS = 500
N_PER = 20
TOTAL = N_GRAPHS * N_PER
E = 320000
D = 128
HD = 128
L = 4
NUM_HEADS = 4
N_ACT = 56
EPS = 1e-5

GP = 512
NP = 10112
NPAD_ROWS = NP - TOTAL
NC = 2
NS = 16
NW = NC * NS
CHUNK = 128
CH = 81
E_PAD = NW * CH * CHUNK
ROWS_PER_TEC = NP // NS

_HIGH = jax.lax.Precision.HIGHEST


def _relu(v):
    return jnp.maximum(v, 0.0)


def _dot(a, b):
    return jax.lax.dot_general(a, b, (((1,), (0,)), ((), ())),
                               preferred_element_type=jnp.float32,
                               precision=_HIGH)


def _zero_block(ref, rows, width):
    z = jnp.zeros((16,), jnp.float32)

    def body(r, _):
        for q in range(width // 16):
            ref[r, pl.ds(q * 16, 16)] = z
        return 0

    lax.fori_loop(0, rows, body, 0)


def _zero_rows(zsrc, dst_sh, my_rows):
    nfull = ROWS_PER_TEC // CHUNK
    rem = ROWS_PER_TEC - nfull * CHUNK
    for k in range(nfull):
        pltpu.sync_copy(zsrc, dst_sh.at[pl.ds(my_rows + k * CHUNK, CHUNK)])
    if rem:
        pltpu.sync_copy(
            zsrc.at[pl.ds(0, rem)],
            dst_sh.at[pl.ds(my_rows + nfull * CHUNK, rem)])


def _make_sc_pass():
    outs = [jax.ShapeDtypeStruct((NC * NP, D), jnp.float32)]
    scratch = [
        [pltpu.VMEM((CHUNK,), jnp.int32) for _ in range(3)],
        [pltpu.VMEM((CHUNK,), jnp.int32) for _ in range(3)],
        [pltpu.VMEM((CHUNK, D), jnp.float32) for _ in range(3)],
        pltpu.VMEM_SHARED((NP, D), jnp.float32),
        [pltpu.SemaphoreType.DMA for _ in range(3)],
        [pltpu.SemaphoreType.DMA for _ in range(3)],
    ]

    def body(h_hbm, src_hbm, dst_hbm, agg_out, src_v, dst_v, rows, agg_sh,
             sem_g, sem_s):
        c = lax.axis_index("c")
        s = lax.axis_index("s")
        wid = c * NS + s
        my_rows = s * ROWS_PER_TEC
        ebase = wid * CH * CHUNK

        _zero_block(rows[0], CHUNK, D)
        _zero_rows(rows[0], agg_sh, my_rows)
        plsc.subcore_barrier()

        def load_idx(i, k):
            pltpu.sync_copy(src_hbm.at[pl.ds(ebase + i * CHUNK, CHUNK)],
                            src_v[k])
            pltpu.sync_copy(dst_hbm.at[pl.ds(ebase + i * CHUNK, CHUNK)],
                            dst_v[k])

        def fire_g(k):
            pltpu.async_copy(h_hbm.at[src_v[k]], rows[k], sem_g[k])

        def wait_g(k):
            pltpu.make_async_copy(h_hbm.at[src_v[k]], rows[k],
                                  sem_g[k]).wait()

        def fire_s(k):
            pltpu.async_copy(rows[k], agg_sh.at[dst_v[k]], sem_s[k],
                             add=True)

        def wait_s(k):
            pltpu.make_async_copy(rows[k], agg_sh.at[dst_v[k]],
                                  sem_s[k]).wait()

        load_idx(0, 0)
        fire_g(0)
        load_idx(1, 1)
        fire_g(1)
        wait_g(0)
        fire_s(0)
        load_idx(2, 2)
        fire_g(2)
        wait_g(1)
        fire_s(1)

        def step(t, _):
            for k in range(3):
                i = 3 * t + k
                wait_s(k)
                load_idx(i, k)
                fire_g(k)
                kp = (k + 2) % 3
                wait_g(kp)
                fire_s(kp)
            return 0

        lax.fori_loop(1, CH // 3, step, 0)
        wait_g(2)
        fire_s(2)
        wait_s(0)
        wait_s(1)
        wait_s(2)
        plsc.subcore_barrier()

        base = c * NP + my_rows
        pltpu.sync_copy(agg_sh.at[pl.ds(my_rows, ROWS_PER_TEC)],
                        agg_out.at[pl.ds(base, ROWS_PER_TEC)])

    mesh = plsc.VectorSubcoreMesh(core_axis_name="c", subcore_axis_name="s",
                                  num_cores=NC, num_subcores=NS)
    return pl.kernel(body, out_type=outs, mesh=mesh, scratch_types=scratch)


DW = 128


def _make_sc_deg():
    outs = [jax.ShapeDtypeStruct((NC * NP, DW), jnp.float32)]
    scratch = [
        pltpu.VMEM((CHUNK,), jnp.int32),
        pltpu.VMEM((CHUNK,), jnp.int32),
        pltpu.VMEM((CHUNK, DW), jnp.float32),
        pltpu.VMEM_SHARED((NP, DW), jnp.float32),
        pltpu.SemaphoreType.DMA,
        pltpu.SemaphoreType.DMA,
    ]

    def body(dst_hbm, degb_out, dst0, dst1, ones_v, deg_sh, sem_a, sem_b):
        c = lax.axis_index("c")
        s = lax.axis_index("s")
        wid = c * NS + s
        my_rows = s * ROWS_PER_TEC
        ebase = wid * CH * CHUNK

        _zero_block(ones_v, CHUNK, DW)
        _zero_rows(ones_v, deg_sh, my_rows)
        one = jnp.ones((16,), jnp.float32)

        def fill_ones(r, _):
            for q in range(DW // 16):
                ones_v[r, pl.ds(q * 16, 16)] = one
            return 0

        lax.fori_loop(0, CHUNK, fill_ones, 0)
        plsc.subcore_barrier()

        pltpu.sync_copy(dst_hbm.at[pl.ds(ebase, CHUNK)], dst0)
        pltpu.async_copy(ones_v, deg_sh.at[dst0], sem_a, add=True)

        def step(t, _):
            i = 2 * t
            pltpu.sync_copy(dst_hbm.at[pl.ds(ebase + (i + 1) * CHUNK, CHUNK)],
                            dst1)
            pltpu.async_copy(ones_v, deg_sh.at[dst1], sem_b, add=True)
            pltpu.make_async_copy(ones_v, deg_sh.at[dst0], sem_a).wait()
            pltpu.sync_copy(dst_hbm.at[pl.ds(ebase + (i + 2) * CHUNK, CHUNK)],
                            dst0)
            pltpu.async_copy(ones_v, deg_sh.at[dst0], sem_a, add=True)
            pltpu.make_async_copy(ones_v, deg_sh.at[dst1], sem_b).wait()
            return 0

        lax.fori_loop(0, CH // 2, step, 0)
        pltpu.make_async_copy(ones_v, deg_sh.at[dst0], sem_a).wait()
        plsc.subcore_barrier()

        base = c * NP + my_rows
        pltpu.sync_copy(deg_sh.at[pl.ds(my_rows, ROWS_PER_TEC)],
                        degb_out.at[pl.ds(base, ROWS_PER_TEC)])

    mesh = plsc.VectorSubcoreMesh(core_axis_name="c", subcore_axis_name="s",
                                  num_cores=NC, num_subcores=NS)
    return pl.kernel(body, out_type=outs, mesh=mesh, scratch_types=scratch)


@functools.cache
def _get_sc_pass():
    return _make_sc_pass()


@functools.cache
def _get_sc_deg():
    return _make_sc_deg()


def _sc_pass(h, src_slab, dst_slab):
    out = _get_sc_pass()(h, src_slab, dst_slab)
    return out[0] if isinstance(out, (list, tuple)) else out


def _sc_deg(dst_slab):
    out = _get_sc_deg()(dst_slab)
    return out[0] if isinstance(out, (list, tuple)) else out


_BLK = 1264
_GRID = NP // _BLK

_row_spec = pl.BlockSpec((_BLK, D), lambda i: (i, 0))
_half0_spec = pl.BlockSpec((_BLK, D), lambda i: (i, 0))
_half1_spec = pl.BlockSpec((_BLK, D), lambda i: (i + _GRID, 0))
_w_spec = pl.BlockSpec((D, D), lambda i: (0, 0))
_b_spec = pl.BlockSpec((1, D), lambda i: (0, 0))


def _proj_body(x_ref, w_ref, b_ref, o_ref):
    o_ref[...] = _relu(_dot(x_ref[...], w_ref[...]) + b_ref[...])


_proj = pl.pallas_call(
    _proj_body,
    grid=(_GRID,),
    in_specs=[_row_spec, _w_spec, _b_spec],
    out_specs=_row_spec,
    out_shape=jax.ShapeDtypeStruct((NP, D), jnp.float32),
)


def _layer1_body(a0_ref, a1_ref, d0_ref, d1_ref, h_ref, w_ref, b_ref,
                 o_ref, rec_ref):
    rec64 = 1.0 / jnp.maximum(d0_ref[...] + d1_ref[...], 1.0)
    rec = jnp.broadcast_to(rec64[:, :1], (_BLK, D))
    rec_ref[...] = rec
    a = (a0_ref[...] + a1_ref[...]) * rec
    o_ref[...] = _relu(_dot(a, w_ref[...]) + b_ref[...]) + h_ref[...]


_deg0_spec = pl.BlockSpec((_BLK, DW), lambda i: (i, 0))
_deg1_spec = pl.BlockSpec((_BLK, DW), lambda i: (i + _GRID, 0))

_layer1 = pl.pallas_call(
    _layer1_body,
    grid=(_GRID,),
    in_specs=[_half0_spec, _half1_spec, _deg0_spec, _deg1_spec, _row_spec,
              _w_spec, _b_spec],
    out_specs=[_row_spec, _row_spec],
    out_shape=[jax.ShapeDtypeStruct((NP, D), jnp.float32),
               jax.ShapeDtypeStruct((NP, D), jnp.float32)],
)


def _layer_body(a0_ref, a1_ref, rec_ref, h_ref, w_ref, b_ref, o_ref):
    a = (a0_ref[...] + a1_ref[...]) * rec_ref[...]
    o_ref[...] = _relu(_dot(a, w_ref[...]) + b_ref[...]) + h_ref[...]


_layer = pl.pallas_call(
    _layer_body,
    grid=(_GRID,),
    in_specs=[_half0_spec, _half1_spec, _row_spec, _row_spec, _w_spec,
              _b_spec],
    out_specs=_row_spec,
    out_shape=jax.ShapeDtypeStruct((NP, D), jnp.float32),
)


def _heads_body(h3_ref, we_ref, be_ref, lng_ref, lnb_ref, wh_ref, bh_ref,
                wp1_ref, bp1_ref, wp2_ref, bp2_ref, wv1_ref, bv1_ref,
                wv2_ref, bv2_ref, l0_ref, l1_ref, l2_ref, l3_ref, v_ref):
    hb = h3_ref[...]
    be = be_ref[...]
    lng = lng_ref[...]
    lnb = lnb_ref[...]
    bp1 = bp1_ref[...]
    bp2 = bp2_ref[...]

    embs = []
    for i in range(NUM_HEADS):
        base = 4 * i
        acc = hb[:, base, :]
        mx = hb[:, base, :]
        for j in range(1, 8):
            t = hb[:, base + j, :]
            acc = acc + t
            mx = jnp.maximum(mx, t)
        m = acc * (1.0 / 8.0)
        y = (_dot(m, we_ref[i, 0:D, :]) + _dot(mx, we_ref[i, D:2 * D, :])
             + be[i:i + 1, :])
        mu = jnp.mean(y, axis=-1, keepdims=True)
        var = jnp.mean((y - mu) ** 2, axis=-1, keepdims=True)
        y = (y - mu) / jnp.sqrt(var + EPS) * lng[i:i + 1, :] + lnb[i:i + 1, :]
        embs.append(_relu(y))

    glob = (embs[0] + embs[1] + embs[2] + embs[3]) * 0.25
    ctx = _relu(_dot(glob, wh_ref[...]) + bh_ref[...])

    outs = [l0_ref, l1_ref, l2_ref, l3_ref]
    for i in range(NUM_HEADS):
        z = _relu(_dot(embs[i], wp1_ref[i, 0:HD, :])
                  + _dot(ctx, wp1_ref[i, HD:2 * HD, :]) + bp1[i:i + 1, :])
        outs[i][...] = _dot(z, wp2_ref[i]) + bp2[i:i + 1, :]

    pv = hb[:, 0, :]
    for j in range(1, N_PER):
        pv = pv + hb[:, j, :]
    pv = pv * (1.0 / N_PER)
    v = _relu(_dot(pv, wv1_ref[...]) + bv1_ref[...])
    v_ref[...] = jnp.tanh(_dot(v, wv2_ref[...]) + bv2_ref[...])


def _full(shape):
    return pl.BlockSpec(shape, lambda: tuple(0 for _ in shape))


_heads = pl.pallas_call(
    _heads_body,
    in_specs=[_full((GP, N_PER, D)), _full((NUM_HEADS, 2 * D, HD)),
              _full((NUM_HEADS, HD)), _full((NUM_HEADS, HD)),
              _full((NUM_HEADS, HD)), _full((HD, HD)), _full((1, HD)),
              _full((NUM_HEADS, 2 * HD, HD)), _full((NUM_HEADS, HD)),
              _full((NUM_HEADS, HD, N_ACT)), _full((NUM_HEADS, N_ACT)),
              _full((D, D)), _full((1, D)), _full((D, 1)), _full((1, 1))],
    out_specs=[_full((GP, N_ACT)), _full((GP, N_ACT)), _full((GP, N_ACT)),
               _full((GP, N_ACT)), _full((GP, 1))],
    out_shape=[jax.ShapeDtypeStruct((GP, N_ACT), jnp.float32)] * NUM_HEADS
    + [jax.ShapeDtypeStruct((GP, 1), jnp.float32)],
)


def kernel(x, edge_index, batch, W_in, b_in, W_layers, b_layers, W_ext, b_ext,
           ln_g, ln_b, W_hub, b_hub, W_p1, b_p1, W_p2, b_p2, W_v1, b_v1,
           W_v2, b_v2):
    del batch

    src = edge_index[0]
    dst = edge_index[1]
    pad = TOTAL + (jnp.arange(E_PAD - E, dtype=jnp.int32) % NPAD_ROWS)
    src_slab = jnp.concatenate([src, pad])
    dst_slab = jnp.concatenate([dst, pad])

    x_pad = jnp.pad(x, ((0, NP - TOTAL), (0, 0)))
    b_in2 = b_in.reshape(1, D)

    h = _proj(x_pad, W_in, b_in2)

    degb = _sc_deg(dst_slab)
    aggs = _sc_pass(h, src_slab, dst_slab)
    h, rec = _layer1(aggs, aggs, degb, degb, h,
                     W_layers[0], b_layers[0].reshape(1, D))
    for l in range(1, L):
        aggs = _sc_pass(h, src_slab, dst_slab)
        h = _layer(aggs, aggs, rec, h, W_layers[l],
                   b_layers[l].reshape(1, D))

    h3 = jnp.pad(h, ((0, GP * N_PER - NP), (0, 0))).reshape(GP, N_PER, D)
    l0, l1, l2, l3, v = _heads(
        h3, W_ext, b_ext, ln_g, ln_b, W_hub, b_hub.reshape(1, HD),
        W_p1, b_p1, W_p2, b_p2, W_v1, b_v1.reshape(1, D), W_v2,
        b_v2.reshape(1, 1))

    return jnp.concatenate(
        [l0[:N_GRAPHS], l1[:N_GRAPHS], l2[:N_GRAPHS], l3[:N_GRAPHS],
         v[:N_GRAPHS]], axis=-1)
